# Initial kernel scaffold; baseline (speedup 1.0000x reference)
import jax
import jax.numpy as jnp
from jax.experimental import pallas as pl


def kernel(h, W0, al0, ar0, b0, W1, al1, ar1, b1, W2, al2, ar2, b2, beta, edge_index0, edge_index1, edge_index2):
    raise NotImplementedError("write your pallas kernel here")


# speedup vs baseline: 25.6753x; 25.6753x over previous
import functools

import jax
import jax.numpy as jnp
import numpy as np
from jax import lax
from jax.experimental import pallas as pl
from jax.experimental.pallas import tpu as pltpu
from jax.experimental.pallas import tpu_sc as plsc

N = 10000
E = 320000
IN = 128
H = 8
F = 32
M = 3

NC = 2
NS = 16
L = 16
HH = H // NC
CW = HH * F
EPT = E // NS
CE = 80
NCHUNK = EPT // CE
CWP = CW + 16
TW = 16
TBS = 400
TPW = (N * TW) // NS
WPT = 640
WLAST = N - (NS - 1) * WPT

_BN = 1000
_NB = N // _BN


def _proj_body(h_ref, w_ref, al_ref, ar_ref, feat_ref, el_ref, er_ref):
  hb = h_ref[...]
  wb = w_ref[0, 0]
  feat = jnp.dot(hb, wb, preferred_element_type=jnp.float32)
  feat_ref[0, 0] = feat
  el_ref[0, 0] = jnp.dot(feat, al_ref[0, 0], preferred_element_type=jnp.float32,
                         precision=lax.Precision.HIGHEST)
  er_ref[0, 0] = jnp.dot(feat, ar_ref[0, 0], preferred_element_type=jnp.float32,
                         precision=lax.Precision.HIGHEST)


@jax.jit
def _project(h, Ws, ALs, ARs):
  grid = (M, NC, _NB)
  return pl.pallas_call(
      _proj_body,
      grid=grid,
      in_specs=[
          pl.BlockSpec((_BN, IN), lambda i, c, j: (j, 0)),
          pl.BlockSpec((1, 1, IN, CW), lambda i, c, j: (i, c, 0, 0)),
          pl.BlockSpec((1, 1, CW, TW), lambda i, c, j: (i, c, 0, 0)),
          pl.BlockSpec((1, 1, CW, TW), lambda i, c, j: (i, c, 0, 0)),
      ],
      out_specs=[
          pl.BlockSpec((1, 1, _BN, CW), lambda i, c, j: (i, c, j, 0)),
          pl.BlockSpec((1, 1, _BN, TW), lambda i, c, j: (i, c, j, 0)),
          pl.BlockSpec((1, 1, _BN, TW), lambda i, c, j: (i, c, j, 0)),
      ],
      out_shape=[
          jax.ShapeDtypeStruct((M, NC, N, CW), jnp.float32),
          jax.ShapeDtypeStruct((M, NC, N, TW), jnp.float32),
          jax.ShapeDtypeStruct((M, NC, N, TW), jnp.float32),
      ],
  )(h, Ws, ALs, ARs)


def _sc_body(els2, ers2, elsf, ersf, feats, e0, e1, e2,
             acc_out,
             src_v, dst_v, idx_v, idx3_v, eeq_v, elr_v, err_v, rows_v,
             rows2_v, tb_v, u_v, u2_v,
             umax_sh, acc_sh, gsem, lsem, rsem, ssem):
  cid = lax.axis_index("c")
  sid = lax.axis_index("s")
  iot = lax.iota(jnp.int32, L)
  lane4 = iot & 3
  quad4 = iot >> 2
  zed = jnp.zeros((L,), jnp.float32)

  edges = [e0, e1, e2]
  for i in range(M):
    row_off = (i * NC + cid) * N
    tab16 = row_off * TW

    def _scan_table(tab, off, nchunks):
      def _body(t, mv):
        pltpu.sync_copy(tab.at[pl.ds(off + t * TBS, TBS)], tb_v)
        def _fold(k, m):
          return jnp.maximum(m, tb_v[pl.ds(k * L, L)])
        return lax.fori_loop(0, TBS // L, _fold, mv)
      return lax.fori_loop(0, nchunks, _body, jnp.full((L,), -3.0e38))

    off = tab16 + sid * TPW
    mv = (_scan_table(elsf, off, TPW // TBS)
          + _scan_table(ersf, off, TPW // TBS))
    u_v[...] = mv
    pltpu.sync_copy(u_v, umax_sh.at[pl.ds(sid * L, L)])

    def _zrow(r, c):
      for j in range(CWP // L):
        rows2_v[r, pl.ds(j * L, L)] = zed
      return c
    lax.fori_loop(0, CE, _zrow, 0)
    base_row = sid * WPT

    def _zero_slices(nrows):
      for t in range(nrows // CE):
        pltpu.sync_copy(rows2_v, acc_sh.at[pl.ds(base_row + t * CE, CE)])

    @pl.when(sid < NS - 1)
    def _():
      _zero_slices(WPT)

    @pl.when(sid == NS - 1)
    def _():
      _zero_slices(WLAST)

    plsc.subcore_barrier()

    pltpu.sync_copy(umax_sh, u2_v)
    def _maxu(k, mv):
      return jnp.maximum(mv, u2_v[pl.ds(k * L, L)])
    mu = lax.fori_loop(0, NS, _maxu, jnp.full((L,), -3.0e38))
    u2_v[pl.ds(0, L)] = mu
    mu = jnp.maximum(mu, plsc.load_gather(u2_v, [iot ^ 8]))
    u2_v[pl.ds(0, L)] = mu
    mu = jnp.maximum(mu, plsc.load_gather(u2_v, [iot ^ 4]))
    U = jnp.where(mu >= 0.0, mu, 0.2 * mu)

    ebase = sid * EPT

    def _chunk(k, carry):
      base = ebase + k * CE
      pltpu.sync_copy(edges[i].at[pl.ds(base, CE)], src_v)
      pltpu.sync_copy(edges[i].at[pl.ds(E + base, CE)], dst_v)

      def _mkidx(m, c):
        idx_v[pl.ds(m * L, L)] = src_v[pl.ds(m * L, L)] + row_off
        idx3_v[pl.ds(m * L, L)] = dst_v[pl.ds(m * L, L)] + row_off
        return c
      lax.fori_loop(0, CE // L, _mkidx, 0)
      gcopy = pltpu.async_copy(feats.at[idx_v], rows_v, gsem)
      lcopy = pltpu.async_copy(els2.at[idx_v], elr_v, lsem)
      rcopy = pltpu.async_copy(ers2.at[idx3_v], err_v, rsem)
      lcopy.wait()
      rcopy.wait()

      def _quad(q, c):
        eidx = 4 * q + quad4
        elv = plsc.load_gather(elr_v, [eidx, lane4])
        erv = plsc.load_gather(err_v, [eidx, lane4])
        x = elv + erv
        e = jnp.where(x >= 0.0, x, 0.2 * x)
        ee = jnp.exp(e - U)
        eeq_v[pl.ds(q * L, L)] = ee
        return c
      lax.fori_loop(0, CE // 4, _quad, 0)

      gcopy.wait()

      def _scale(q, c):
        v16 = eeq_v[pl.ds(q * L, L)]
        for kk in range(4):
          row = 4 * q + kk
          for hh in range(HH):
            m16 = jnp.broadcast_to(v16[4 * kk + hh], (L,))
            for j2 in (2 * hh, 2 * hh + 1):
              rows2_v[row, pl.ds(j2 * L, L)] = (
                  rows_v[row, pl.ds(j2 * L, L)] * m16)
          eo = plsc.load_gather(eeq_v, [4 * row + lane4])
          rows2_v[row, pl.ds(CW, L)] = jnp.where(iot < HH, eo, 0.0)
        return c
      lax.fori_loop(0, CE // 4, _scale, 0)

      scopy = pltpu.async_copy(rows2_v, acc_sh.at[dst_v], ssem, add=True)
      scopy.wait()
      return carry

    lax.fori_loop(0, NCHUNK, _chunk, 0)

    plsc.subcore_barrier()

    out_row = row_off + base_row

    def _writeout(nrows):
      pltpu.sync_copy(acc_sh.at[pl.ds(base_row, nrows)],
                      acc_out.at[pl.ds(out_row, nrows)])

    @pl.when(sid < NS - 1)
    def _():
      _writeout(WPT)

    @pl.when(sid == NS - 1)
    def _():
      _writeout(WLAST)


@jax.jit
def _sc_message(els2, ers2, elsf, ersf, feats, e0, e1, e2):
  mesh = plsc.VectorSubcoreMesh(
      core_axis_name="c", subcore_axis_name="s", num_cores=NC, num_subcores=NS)
  fn = pl.kernel(
      _sc_body,
      out_type=jax.ShapeDtypeStruct((M * NC * N, CWP), jnp.float32),
      mesh=mesh,
      compiler_params=pltpu.CompilerParams(
          needs_layout_passes=False, use_tc_tiling_on_sc=False),
      scratch_types=[
          pltpu.VMEM((CE,), jnp.int32),
          pltpu.VMEM((CE,), jnp.int32),
          pltpu.VMEM((CE,), jnp.int32),
          pltpu.VMEM((CE,), jnp.int32),
          pltpu.VMEM((CE * HH,), jnp.float32),
          pltpu.VMEM((CE, TW), jnp.float32),
          pltpu.VMEM((CE, TW), jnp.float32),
          pltpu.VMEM((CE, CW), jnp.float32),
          pltpu.VMEM((CE, CWP), jnp.float32),
          pltpu.VMEM((TBS,), jnp.float32),
          pltpu.VMEM((L,), jnp.float32),
          pltpu.VMEM((NS * L,), jnp.float32),
          pltpu.VMEM_SHARED((NS * L,), jnp.float32),
          pltpu.VMEM_SHARED((N, CWP), jnp.float32),
          pltpu.SemaphoreType.DMA,
          pltpu.SemaphoreType.DMA,
          pltpu.SemaphoreType.DMA,
          pltpu.SemaphoreType.DMA,
      ],
  )
  return fn(els2, ers2, elsf, ersf, feats, e0, e1, e2)


def _comb_body(acc_ref, rep_ref, beta_ref, bias_ref, out_ref):
  i = pl.program_id(2)
  ab = acc_ref[0, 0]
  d4 = ab[:, CW:CW + HH]
  rden = jnp.where(d4 > 0.0, 1.0 / d4, 1.0)
  r128 = jnp.dot(rden, rep_ref[...], preferred_element_type=jnp.float32)
  z = ab[:, 0:CW] * r128 + bias_ref[0, 0]
  ez = jnp.where(z > 0.0, z, jnp.exp(jnp.minimum(z, 0.0)) - 1.0)
  contrib = beta_ref[i, 0] * ez

  @pl.when(i == 0)
  def _():
    out_ref[...] = contrib

  @pl.when(i > 0)
  def _():
    out_ref[...] += contrib


@jax.jit
def _combine(acc, rep, beta, bias):
  grid = (_NB, NC, M)
  return pl.pallas_call(
      _comb_body,
      grid=grid,
      in_specs=[
          pl.BlockSpec((1, 1, _BN, CWP), lambda j, c, i: (i, c, j, 0)),
          pl.BlockSpec((HH, CW), lambda j, c, i: (0, 0)),
          pl.BlockSpec(memory_space=pltpu.SMEM),
          pl.BlockSpec((1, 1, 1, CW), lambda j, c, i: (i, c, 0, 0)),
      ],
      out_specs=pl.BlockSpec((_BN, CW), lambda j, c, i: (j, c)),
      out_shape=jax.ShapeDtypeStruct((N, H * F), jnp.float32),
  )(acc, rep, beta, bias)


def kernel(h, W0, al0, ar0, b0, W1, al1, ar1, b1, W2, al2, ar2, b2, beta,
           edge_index0, edge_index1, edge_index2):
  Ws = jnp.stack([W0, W1, W2])
  Ws = Ws.reshape(M, IN, NC, CW).transpose(0, 2, 1, 3)

  K = jnp.kron(jnp.eye(HH, dtype=jnp.float32), jnp.ones((F, 1), jnp.float32))
  als = jnp.stack([al0, al1, al2]).reshape(M, H * F)
  ars = jnp.stack([ar0, ar1, ar2]).reshape(M, H * F)
  ALs = als.reshape(M, NC, CW)[..., None] * K
  ARs = ars.reshape(M, NC, CW)[..., None] * K
  pad = jnp.zeros((M, NC, CW, TW - HH), jnp.float32)
  ALs = jnp.concatenate([ALs, pad], axis=-1)
  ARs = jnp.concatenate([ARs, pad], axis=-1)

  feat, el, er = _project(h, Ws, ALs, ARs)
  els2 = el.reshape(M * NC * N, TW)
  ers2 = er.reshape(M * NC * N, TW)
  elsf = el.reshape(M * NC * N * TW)
  ersf = er.reshape(M * NC * N * TW)
  feats = feat.reshape(M * NC * N, CW)

  acc = _sc_message(els2, ers2, elsf, ersf, feats,
                    edge_index0.reshape(2 * E),
                    edge_index1.reshape(2 * E),
                    edge_index2.reshape(2 * E))

  rep = jnp.kron(jnp.eye(HH, dtype=jnp.float32), jnp.ones((1, F), jnp.float32))
  bias = jnp.stack([b0, b1, b2]).reshape(M, NC, 1, CW)
  out = _combine(acc.reshape(M, NC, N, CWP), rep, beta, bias)
  return out
